# baseline (speedup 1.0000x reference)
import jax
import jax.numpy as jnp
from jax.experimental import pallas as pl
from jax.experimental.pallas import tpu as pltpu

NBITS = 8192
KWIN = 10
BLK = 256
NBLKS = NBITS // BLK


def _fc_body(x_ref, w_ref, b_ref, o_ref, acc_ref, cand_ref):
    i = pl.program_id(0)
    part = jax.lax.dot_general(
        x_ref[...], w_ref[...],
        dimension_numbers=(((1,), (1,)), ((), ())),
        preferred_element_type=jnp.float32,
    ) + b_ref[...]
    acc_ref[:, pl.ds(i * BLK, BLK)] = part

    @pl.when(i == 0)
    def _():
        cand_ref[...] = jnp.full((KWIN, BLK), -jnp.inf, jnp.float32)

    @pl.when(i < NBLKS - 1)
    def _():
        v = part
        for t in range(KWIN):
            c = cand_ref[pl.ds(t, 1), :]
            hi = jnp.maximum(c, v)
            v = jnp.minimum(c, v)
            cand_ref[pl.ds(t, 1), :] = hi

    @pl.when(i == NBLKS - 1)
    def _():
        cand = cand_ref[...]

        pvals = []
        m = jnp.float32(jnp.inf)
        for _ in range(KWIN):
            m = jnp.max(jnp.where(cand < m, cand, -jnp.inf))
            pvals.append(m)
        pcnts = [jnp.sum((cand == v).astype(jnp.int32)) for v in pvals]

        vals = []
        m = jnp.float32(jnp.inf)
        for _ in range(KWIN):
            mp = jnp.max(jnp.where(part < m, part, -jnp.inf))
            ms = jnp.float32(-jnp.inf)
            for pv in pvals:
                ms = jnp.maximum(ms, jnp.where(pv < m, pv, -jnp.inf))
            m = jnp.maximum(mp, ms)
            vals.append(m)
        cnts = [
            jnp.sum((part == v).astype(jnp.int32))
            + sum(
                jnp.where(pv == v, pc, 0)
                for pv, pc in zip(pvals, pcnts)
            )
            for v in vals
        ]
        thr = vals[0]
        cum = cnts[0]
        for r in range(1, KWIN):
            need = cum < KWIN
            thr = jnp.where(need, vals[r], thr)
            cum = jnp.where(need, cum + cnts[r], cum)

        out = acc_ref[...]
        o_ref[...] = jnp.where(out >= thr, out, 0.0)


def kernel(x, W, b):
    b_row = b.reshape(1, NBITS)
    return pl.pallas_call(
        _fc_body,
        grid=(NBLKS,),
        in_specs=[
            pl.BlockSpec((1, NBITS), lambda i: (0, 0)),
            pl.BlockSpec((BLK, NBITS), lambda i: (i, 0)),
            pl.BlockSpec((1, BLK), lambda i: (0, i)),
        ],
        out_specs=pl.BlockSpec((1, NBITS), lambda i: (0, 0)),
        out_shape=jax.ShapeDtypeStruct((1, NBITS), jnp.float32),
        scratch_shapes=[
            pltpu.VMEM((1, NBITS), jnp.float32),
            pltpu.VMEM((KWIN, BLK), jnp.float32),
        ],
    )(x, W, b_row)


# speedup vs baseline: 1.0381x; 1.0381x over previous
import jax
import jax.numpy as jnp
from jax.experimental import pallas as pl
from jax.experimental.pallas import tpu as pltpu

NBITS = 8192
KWIN = 10
BLK = 256
NBLKS = NBITS // BLK


def _fc_body(x_ref, w_ref, b_ref, o_ref, acc_ref, cand_ref):
    i = pl.program_id(0)
    part = jax.lax.dot_general(
        x_ref[...], w_ref[...],
        dimension_numbers=(((1,), (1,)), ((), ())),
        preferred_element_type=jnp.float32,
    ) + b_ref[...]
    acc_ref[:, pl.ds(i * BLK, BLK)] = part

    @pl.when(i == 0)
    def _():
        cand_ref[...] = jnp.full((KWIN, BLK), -jnp.inf, jnp.float32)

    v = part
    for t in range(KWIN):
        c = cand_ref[pl.ds(t, 1), :]
        hi = jnp.maximum(c, v)
        v = jnp.minimum(c, v)
        cand_ref[pl.ds(t, 1), :] = hi

    @pl.when(i == NBLKS - 1)
    def _():
        cand = cand_ref[...]

        vals = []
        m = jnp.float32(jnp.inf)
        for _ in range(KWIN):
            m = jnp.max(jnp.where(cand < m, cand, -jnp.inf))
            vals.append(m)
        cnts = [jnp.sum((cand == v).astype(jnp.int32)) for v in vals]
        thr = vals[0]
        cum = cnts[0]
        for r in range(1, KWIN):
            need = cum < KWIN
            thr = jnp.where(need, vals[r], thr)
            cum = jnp.where(need, cum + cnts[r], cum)

        out = acc_ref[...]
        o_ref[...] = jnp.where(out >= thr, out, 0.0)


def kernel(x, W, b):
    b_row = b.reshape(1, NBITS)
    return pl.pallas_call(
        _fc_body,
        grid=(NBLKS,),
        in_specs=[
            pl.BlockSpec((1, NBITS), lambda i: (0, 0)),
            pl.BlockSpec((BLK, NBITS), lambda i: (i, 0)),
            pl.BlockSpec((1, BLK), lambda i: (0, i)),
        ],
        out_specs=pl.BlockSpec((1, NBITS), lambda i: (0, 0)),
        out_shape=jax.ShapeDtypeStruct((1, NBITS), jnp.float32),
        scratch_shapes=[
            pltpu.VMEM((1, NBITS), jnp.float32),
            pltpu.VMEM((KWIN, BLK), jnp.float32),
        ],
    )(x, W, b_row)
